# baseline (speedup 1.0000x reference)
import dataclasses
import functools

import jax
import jax.numpy as jnp
from jax import lax
from jax.experimental import pallas as pl
from jax.experimental.pallas import tpu as pltpu
from jax.experimental.pallas import tpu_sc as plsc

_GATHER_WINDOW = 128
_GATHER_K = 8


def _sc_compiler_params():
    cp = pltpu.CompilerParams(use_tc_tiling_on_sc=False)
    if "needs_layout_passes" in pltpu.CompilerParams.__dataclass_fields__:
        cp = dataclasses.replace(cp, needs_layout_passes=False)
    return cp


def _sc_gather_t(table, ids2):
    num = ids2.shape[1]
    depth = table.shape[1]
    step_cols = _GATHER_WINDOW * _GATHER_K
    n_tiles = step_cols // 16
    mesh = plsc.VectorSubcoreMesh(core_axis_name="c", subcore_axis_name="s")

    @functools.partial(
        pl.kernel,
        out_type=jax.ShapeDtypeStruct((depth, num), table.dtype),
        mesh=mesh,
        scratch_types=[
            pltpu.VMEM((step_cols, depth), table.dtype),
            pltpu.SemaphoreType.DMA,
        ],
        compiler_params=_sc_compiler_params(),
    )
    def gather_kernel(table_hbm, ids_hbm, out_hbm, rows_v, sem):
        def body(i_vmem, o_vmem):
            copies = []
            for j in range(_GATHER_K):
                sl = pl.ds(j * _GATHER_WINDOW, _GATHER_WINDOW)
                copies.append(
                    pltpu.make_async_copy(
                        table_hbm.at[i_vmem.at[0, sl]], rows_v.at[sl], sem
                    )
                )
            for c in copies:
                c.start()
            for c in copies:
                c.wait()

            lanes = lax.iota(jnp.int32, 16)

            @pl.loop(0, n_tiles)
            def _(t):
                @pl.loop(0, depth)
                def _(k):
                    v = plsc.load_gather(rows_v, [t * 16 + lanes, lanes * 0 + k])
                    o_vmem[k, pl.ds(t * 16, 16)] = v

        pltpu.emit_pipeline(
            body,
            grid=(num // step_cols,),
            in_specs=[
                pl.BlockSpec((1, step_cols), lambda i: (0, i)),
            ],
            out_specs=[
                pl.BlockSpec((depth, step_cols), lambda i: (0, i)),
            ],
            core_axis_name=("c", "s"),
            dimension_semantics=(pltpu.PARALLEL,),
        )(ids_hbm, out_hbm)

    return gather_kernel(table, ids2)


def _mlp_t(ea_t, emb_t, w1eT, w1nT, b1c, w2T, b2c, block_cols):
    feat, num = ea_t.shape
    depth = emb_t.shape[0]
    hid = w2T.shape[0]

    def body(ea_ref, emb_ref, w1e_ref, w1n_ref, b1_ref, w2_ref, b2_ref, o_ref):
        h = jnp.dot(w1e_ref[...], emb_ref[...],
                    preferred_element_type=jnp.float32)
        h = h + jnp.dot(w1n_ref[...], ea_ref[...],
                        preferred_element_type=jnp.float32)
        h = jnp.maximum(h + b1_ref[...], 0.0)
        o_ref[...] = (
            jnp.dot(w2_ref[...], h, preferred_element_type=jnp.float32)
            + b2_ref[...]
        )

    return pl.pallas_call(
        body,
        grid=(num // block_cols,),
        in_specs=[
            pl.BlockSpec((feat, block_cols), lambda i: (0, i)),
            pl.BlockSpec((depth, block_cols), lambda i: (0, i)),
            pl.BlockSpec((hid, depth), lambda i: (0, 0)),
            pl.BlockSpec((hid, feat), lambda i: (0, 0)),
            pl.BlockSpec((hid, 1), lambda i: (0, 0)),
            pl.BlockSpec((hid, hid), lambda i: (0, 0)),
            pl.BlockSpec((hid, 1), lambda i: (0, 0)),
        ],
        out_specs=pl.BlockSpec((hid, block_cols), lambda i: (0, i)),
        out_shape=jax.ShapeDtypeStruct((hid, num), jnp.float32),
    )(ea_t, emb_t, w1eT, w1nT, b1c, w2T, b2c)


def kernel(edge_attr, table, W1, b1, W2, b2):
    num = edge_attr.shape[0]
    depth = table.shape[1]
    hid = W1.shape[1]
    ea_t = edge_attr.T
    ids2 = ea_t[0:1, :].astype(jnp.int32)
    emb_t = _sc_gather_t(table, ids2)
    w1eT = W1[:depth].T
    w1nT = jnp.concatenate(
        [jnp.zeros((1, hid), W1.dtype), W1[depth:]], axis=0
    ).T
    b1c = b1.reshape(hid, 1)
    b2c = b2.reshape(hid, 1)
    out_t = _mlp_t(ea_t, emb_t, w1eT, w1nT, b1c, w2T=W2.T, b2c=b2c,
                   block_cols=6400)
    return out_t.T


# speedup vs baseline: 2.7768x; 2.7768x over previous
import dataclasses
import functools

import jax
import jax.numpy as jnp
from jax.experimental import pallas as pl
from jax.experimental.pallas import tpu as pltpu
from jax.experimental.pallas import tpu_sc as plsc

_GATHER_WINDOW = 128
_GATHER_K = 8


def _sc_compiler_params():
    cp = pltpu.CompilerParams(use_tc_tiling_on_sc=False)
    if "needs_layout_passes" in pltpu.CompilerParams.__dataclass_fields__:
        cp = dataclasses.replace(cp, needs_layout_passes=False)
    return cp


def _sc_gather_2d(table, ids2):
    num = ids2.shape[1]
    depth = table.shape[1]
    step_rows = _GATHER_WINDOW * _GATHER_K
    mesh = plsc.VectorSubcoreMesh(core_axis_name="c", subcore_axis_name="s")

    @functools.partial(
        pl.kernel,
        out_type=jax.ShapeDtypeStruct((num, depth), table.dtype),
        mesh=mesh,
        scratch_types=[pltpu.SemaphoreType.DMA],
        compiler_params=_sc_compiler_params(),
    )
    def gather_kernel(table_hbm, ids_hbm, out_hbm, sem):
        def body(i_vmem, o_vmem):
            copies = []
            for j in range(_GATHER_K):
                sl = pl.ds(j * _GATHER_WINDOW, _GATHER_WINDOW)
                copies.append(
                    pltpu.make_async_copy(
                        table_hbm.at[i_vmem.at[0, sl]], o_vmem.at[sl], sem
                    )
                )
            for c in copies:
                c.start()
            for c in copies:
                c.wait()

        pltpu.emit_pipeline(
            body,
            grid=(num // step_rows,),
            in_specs=[
                pl.BlockSpec((1, step_rows), lambda i: (0, i)),
            ],
            out_specs=[
                pl.BlockSpec((step_rows, depth), lambda i: (i, 0)),
            ],
            core_axis_name=("c", "s"),
            dimension_semantics=(pltpu.PARALLEL,),
        )(ids_hbm, out_hbm)

    return gather_kernel(table, ids2)


def _mlp_t(ea_t, emb, w1eT, w1nT, b1c, w2T, b2c, block_cols):
    feat, num = ea_t.shape
    depth = emb.shape[1]
    hid = w2T.shape[0]
    nsteps = num // block_cols

    def body(ea_ref, emb_hbm, w1e_ref, w1n_ref, b1_ref, w2_ref, b2_ref,
             o_ref, buf, sems):
        i = pl.program_id(0)
        slot = jax.lax.rem(i, 2)
        nxt = jax.lax.rem(i + 1, 2)

        def emb_copy(step, b):
            return pltpu.make_async_copy(
                emb_hbm.at[pl.ds(step * block_cols, block_cols), :],
                buf.at[b],
                sems.at[b],
            )

        @pl.when(i == 0)
        def _():
            emb_copy(i, slot).start()

        @pl.when(i + 1 < nsteps)
        def _():
            emb_copy(i + 1, nxt).start()

        emb_copy(i, slot).wait()
        emb_t = jnp.transpose(buf[slot])
        h = jnp.dot(w1e_ref[...], emb_t, preferred_element_type=jnp.float32)
        h = h + jnp.dot(w1n_ref[...], ea_ref[...],
                        preferred_element_type=jnp.float32)
        h = jnp.maximum(h + b1_ref[...], 0.0)
        o_ref[...] = (
            jnp.dot(w2_ref[...], h, preferred_element_type=jnp.float32)
            + b2_ref[...]
        )

    return pl.pallas_call(
        body,
        grid=(nsteps,),
        in_specs=[
            pl.BlockSpec((feat, block_cols), lambda i: (0, i)),
            pl.BlockSpec(memory_space=pl.ANY),
            pl.BlockSpec((hid, depth), lambda i: (0, 0)),
            pl.BlockSpec((hid, feat), lambda i: (0, 0)),
            pl.BlockSpec((hid, 1), lambda i: (0, 0)),
            pl.BlockSpec((hid, hid), lambda i: (0, 0)),
            pl.BlockSpec((hid, 1), lambda i: (0, 0)),
        ],
        out_specs=pl.BlockSpec((hid, block_cols), lambda i: (0, i)),
        out_shape=jax.ShapeDtypeStruct((hid, num), jnp.float32),
        scratch_shapes=[
            pltpu.VMEM((2, block_cols, depth), jnp.float32),
            pltpu.SemaphoreType.DMA((2,)),
        ],
    )(ea_t, emb, w1eT, w1nT, b1c, w2T, b2c)


def kernel(edge_attr, table, W1, b1, W2, b2):
    num = edge_attr.shape[0]
    depth = table.shape[1]
    hid = W1.shape[1]
    ea_t = edge_attr.T
    ids2 = ea_t[0:1, :].astype(jnp.int32)
    emb = _sc_gather_2d(table, ids2)
    w1eT = W1[:depth].T
    w1nT = jnp.concatenate(
        [jnp.zeros((1, hid), W1.dtype), W1[depth:]], axis=0
    ).T
    b1c = b1.reshape(hid, 1)
    b2c = b2.reshape(hid, 1)
    out_t = _mlp_t(ea_t, emb, w1eT, w1nT, b1c, w2T=W2.T, b2c=b2c,
                   block_cols=6400)
    return out_t.T
